# baseline (speedup 1.0000x reference)
import jax
import jax.numpy as jnp
from jax.experimental import pallas as pl
from jax.experimental.pallas import tpu as pltpu

_BUFFER_SIZE = 4194304
_TAIL = 1536
_BUF_LEN = _BUFFER_SIZE + _TAIL
_W = 512
_TROWS = _TAIL // _W
_XROWS = _BUFFER_SIZE // _W
_OROWS = _BUF_LEN // _W
_R = 1024
_NBLK_X = _XROWS // _R
_GRID = _NBLK_X + 1


def _concat_kernel(tail_ref, x_ref, out_ref, carry_ref):
    i = pl.program_id(0)

    @pl.when(i == 0)
    def _():
        out_ref[pl.ds(0, _TROWS), :] = tail_ref[:, :]

    @pl.when(i > 0)
    def _():
        out_ref[pl.ds(0, _TROWS), :] = carry_ref[:, :]

    out_ref[pl.ds(_TROWS, _R - _TROWS), :] = x_ref[pl.ds(0, _R - _TROWS), :]
    carry_ref[:, :] = x_ref[pl.ds(_R - _TROWS, _TROWS), :]


def kernel(x, buffer):
    xv = x.reshape(_XROWS, _W)
    tail = buffer.reshape(_OROWS, _W)[_XROWS:, :]
    out = pl.pallas_call(
        _concat_kernel,
        grid=(_GRID,),
        out_shape=jax.ShapeDtypeStruct((_OROWS, _W), jnp.float32),
        in_specs=[
            pl.BlockSpec((_TROWS, _W), lambda i: (0, 0)),
            pl.BlockSpec((_R, _W), lambda i: (jnp.minimum(i, _NBLK_X - 1), 0)),
        ],
        out_specs=pl.BlockSpec((_R, _W), lambda i: (i, 0)),
        scratch_shapes=[pltpu.VMEM((_TROWS, _W), jnp.float32)],
    )(tail, xv)
    return out.reshape(1, _BUF_LEN)
